# baseline (speedup 1.0000x reference)
import jax
import jax.numpy as jnp
from jax.experimental import pallas as pl
from jax.experimental.pallas import tpu as pltpu


def _dma_copy_body(q_ref, o_ref, sem):
    copy = pltpu.make_async_copy(q_ref, o_ref, sem)
    copy.start()
    copy.wait()


def kernel(c, q, emb_table):
    del c, emb_table
    return pl.pallas_call(
        _dma_copy_body,
        in_specs=[pl.BlockSpec(memory_space=pl.ANY)],
        out_specs=pl.BlockSpec(memory_space=pl.ANY),
        out_shape=jax.ShapeDtypeStruct(q.shape, q.dtype),
        scratch_shapes=[pltpu.SemaphoreType.DMA],
    )(q)


# speedup vs baseline: 12.7064x; 12.7064x over previous
import jax
import jax.numpy as jnp
from jax.experimental import pallas as pl
from jax.experimental.pallas import tpu as pltpu


def _copy_body(q_ref, o_ref):
    o_ref[...] = q_ref[...]


def kernel(c, q, emb_table):
    del c, emb_table
    rows, cols = q.shape
    return pl.pallas_call(
        _copy_body,
        out_shape=jax.ShapeDtypeStruct((rows, cols), q.dtype),
    )(q)


# speedup vs baseline: 38.3906x; 3.0214x over previous
import jax
import jax.numpy as jnp
from jax.experimental import pallas as pl


def _copy_body(q_ref, o_ref):
    o_ref[...] = q_ref[...]


def kernel(c, q, emb_table):
    del c, emb_table
    rows, cols = q.shape
    qt = q.T
    grid = 8
    blk = qt.shape[1] // grid
    out_t = pl.pallas_call(
        _copy_body,
        grid=(grid,),
        in_specs=[pl.BlockSpec((cols, blk), lambda i: (0, i))],
        out_specs=pl.BlockSpec((cols, blk), lambda i: (0, i)),
        out_shape=jax.ShapeDtypeStruct((cols, rows), q.dtype),
    )(qt)
    return out_t.T
